# baseline (speedup 1.0000x reference)
import jax
import jax.numpy as jnp
from jax import lax
from jax.experimental import pallas as pl
from jax.experimental.pallas import tpu as pltpu
from jax.experimental.pallas import tpu_sc as plsc

MAX_DELTA = 24.0

B, F, C, H, W = 8, 2, 3, 512, 512
K = 6
NW = 32
ROWS = H // NW
L = 16
NSLOT = 2
NCLS = 5


def _sc_body(frames_hbm, masks_hbm, raw_hbm, out_hbm, *scratch):
    mask_ring = scratch[0:NSLOT]
    in_ring = [scratch[NSLOT + s * C:NSLOT + (s + 1) * C]
               for s in range(NSLOT)]
    o = NSLOT + NSLOT * C
    out_ring = [scratch[o + s * C:o + (s + 1) * C]
                for s in range(NSLOT)]
    raw_v = scratch[o + NSLOT * C]
    sems = scratch[-1]
    wid = lax.axis_index("s") * 2 + lax.axis_index("c")
    row0 = wid * ROWS

    pltpu.sync_copy(raw_hbm, raw_v)
    tab_vecs = []
    for c in range(C):
        x = raw_v[c]
        a = jnp.abs(x)
        e = jnp.exp(-2.0 * a)
        t = (1.0 - e) / (1.0 + e)
        tab_vecs.append(MAX_DELTA * jnp.sign(x) * t)

    def in_copies(b, slot):
        cps = [pltpu.make_async_copy(
            masks_hbm.at[b, pl.ds(row0, ROWS), :],
            mask_ring[slot], sems.at[slot, 0])]
        for c in range(C):
            cps.append(pltpu.make_async_copy(
                frames_hbm.at[b, 1, c, pl.ds(row0, ROWS), :],
                in_ring[slot][c], sems.at[slot, 1 + c]))
        return cps

    def out_copies(b, slot):
        return [pltpu.make_async_copy(
            out_ring[slot][c], out_hbm.at[b, c, pl.ds(row0, ROWS), :],
            sems.at[slot, 4 + c]) for c in range(C)]

    for cp in in_copies(0, 0):
        cp.start()
    for b in range(K):
        slot = b % NSLOT
        if b + 1 < K:
            for cp in in_copies(b + 1, (b + 1) % NSLOT):
                cp.start()
        for cp in in_copies(b, slot):
            cp.wait()
        if b >= NSLOT:
            for cp in out_copies(b - NSLOT, slot):
                cp.wait()

        def step(r, carry, slot=slot):
            m_row = mask_ring[slot]
            for j in range(W // L):
                m = m_row[r, pl.ds(j * L, L)]
                for c in range(C):
                    f = in_ring[slot][c][r, pl.ds(j * L, L)]
                    d = lax.gather(
                        tab_vecs[c], m[:, None],
                        lax.GatherDimensionNumbers(
                            offset_dims=(), collapsed_slice_dims=(0,),
                            start_index_map=(0,)),
                        slice_sizes=(1,),
                        mode=lax.GatherScatterMode.PROMISE_IN_BOUNDS)
                    r_ = jnp.minimum(jnp.maximum(f + d, 0.0), 255.0)
                    out_ring[slot][c][r, pl.ds(j * L, L)] = r_
            return carry

        lax.fori_loop(0, ROWS, step, 0)
        for cp in out_copies(b, slot):
            cp.start()
    for b in range(K - NSLOT, K):
        for cp in out_copies(b, b % NSLOT):
            cp.wait()


def _sc_update(frames, masks, raw_pad):
    mesh = plsc.VectorSubcoreMesh(core_axis_name="c", subcore_axis_name="s")
    run = pl.kernel(
        _sc_body, mesh=mesh,
        out_type=jax.ShapeDtypeStruct((K, C, H, W), jnp.float32),
        scratch_types=(
            [pltpu.VMEM((ROWS, W), jnp.int32) for _ in range(NSLOT)]
            + [pltpu.VMEM((ROWS, W), jnp.float32) for _ in range(NSLOT * C)]
            + [pltpu.VMEM((ROWS, W), jnp.float32) for _ in range(NSLOT * C)]
            + [pltpu.VMEM((C, L), jnp.float32)]
            + [pltpu.SemaphoreType.DMA((NSLOT, 7))]
        ),
    )
    return run(frames, masks, raw_pad)


def _tc_fill_body(frames_ref, out_ref):
    out_ref[0, 0] = frames_ref[0, 0]


def _tc_fill_ch0(frames):
    return pl.pallas_call(
        _tc_fill_body,
        grid=(B,),
        in_specs=[pl.BlockSpec((1, 1, C, H, W), lambda b: (b, 0, 0, 0, 0))],
        out_specs=pl.BlockSpec((1, 1, C, H, W), lambda b: (b, 0, 0, 0, 0)),
        out_shape=jax.ShapeDtypeStruct((B, F, C, H, W), jnp.float32),
    )(frames)


def _tc_tail_body(frames_ref, masks_ref, delta_ref, _prev_ref, out_ref):
    m = masks_ref[0]
    for c in range(C):
        f = frames_ref[0, 0, c]
        d = delta_ref[NCLS - 1, c]
        for cls in range(NCLS - 2, -1, -1):
            d = jnp.where(m == cls, delta_ref[cls, c], d)
        out_ref[0, 0, c] = jnp.minimum(jnp.maximum(f + d, 0.0), 255.0)


def _tc_lut_tail(frames, masks, delta, prev):
    return pl.pallas_call(
        _tc_tail_body,
        grid=(B - K,),
        in_specs=[
            pl.BlockSpec((1, 1, C, H, W), lambda b: (K + b, 1, 0, 0, 0)),
            pl.BlockSpec((1, H, W), lambda b: (K + b, 0, 0)),
            pl.BlockSpec(memory_space=pltpu.SMEM),
            pl.BlockSpec(memory_space=pl.ANY),
        ],
        out_specs=pl.BlockSpec((1, 1, C, H, W), lambda b: (K + b, 1, 0, 0, 0)),
        out_shape=jax.ShapeDtypeStruct((B, F, C, H, W), jnp.float32),
        input_output_aliases={3: 0},
    )(frames, masks, delta, prev)


def _tc_merge_body(sc_ref, _prev_ref, out_ref):
    out_ref[0, 0] = sc_ref[0]


def _tc_merge(sc_part, prev):
    return pl.pallas_call(
        _tc_merge_body,
        grid=(K,),
        in_specs=[
            pl.BlockSpec((1, C, H, W), lambda b: (b, 0, 0, 0)),
            pl.BlockSpec(memory_space=pl.ANY),
        ],
        out_specs=pl.BlockSpec((1, 1, C, H, W), lambda b: (b, 1, 0, 0, 0)),
        out_shape=jax.ShapeDtypeStruct((B, F, C, H, W), jnp.float32),
        input_output_aliases={1: 0},
    )(sc_part, prev)


def kernel(frames, masks, raw):
    raw_pad = jnp.zeros((C, L), jnp.float32).at[:, :NCLS].set(raw.T)
    delta = MAX_DELTA * jnp.tanh(raw)
    sc_part = _sc_update(frames, masks, raw_pad)
    out = _tc_fill_ch0(frames)
    out = _tc_lut_tail(frames, masks, delta, out)
    return _tc_merge(sc_part, out)
